# baseline (speedup 1.0000x reference)
import math

import jax
import jax.numpy as jnp
from jax.experimental import pallas as pl

H = 16
D = 64
NH = 2
RADIUS = 6
KBAND = 2 * RADIUS + 1
KPAR = 3
KCHILD = 2
KCAND = KBAND + KPAR + KCHILD


def _prep_body(x_ref, g_ref, b_ref, p1_ref, p2_ref, x0_ref):
    xr = x_ref[0]
    p1b = jnp.max(xr, axis=2)
    p2b = jnp.max(p1b, axis=1)
    m = jnp.mean(p1b, axis=-1, keepdims=True)
    v = jnp.mean((p1b - m) ** 2, axis=-1, keepdims=True)
    x0b = (p1b - m) * jax.lax.rsqrt(v + 1e-5) * g_ref[0] + b_ref[0]
    p1_ref[0] = p1b
    p2_ref[0] = p2b
    x0_ref[0] = x0b


def _matmul_body(a_ref, w_ref, o_ref):
    o_ref[...] = jnp.dot(a_ref[...], w_ref[...],
                         preferred_element_type=jnp.float32)


def _matmul(a, w, bm=512):
    m, k = a.shape
    _, n = w.shape
    return pl.pallas_call(
        _matmul_body,
        grid=(m // bm,),
        in_specs=[
            pl.BlockSpec((bm, k), lambda i: (i, 0)),
            pl.BlockSpec((k, n), lambda i: (0, 0)),
        ],
        out_specs=pl.BlockSpec((bm, n), lambda i: (i, 0)),
        out_shape=jax.ShapeDtypeStruct((m, n), jnp.float32),
    )(a.astype(jnp.bfloat16), w.astype(jnp.bfloat16))


def _attn_body(q_ref, k0_ref, v0_ref, kp_ref, vp_ref, kc_ref, vc_ref, o_ref):
    q = q_ref[0]
    l, w = q.shape
    lh = l // 2

    row = jax.lax.broadcasted_iota(jnp.int32, (w, NH), 0)
    col = jax.lax.broadcasted_iota(jnp.int32, (w, NH), 1)
    sel = (row // D == col).astype(jnp.float32)

    def head_sums(prod):
        return jnp.dot(prod, sel, preferred_element_type=jnp.float32)

    cand = []
    for o in range(KBAND):
        cand.append(head_sums(q * k0_ref[0, o:o + l]))
    qp = q.reshape(lh, 2, w)
    for s in (1, 0, 2):
        kd = kp_ref[0, s:s + lh]
        prod = (qp * kd[:, None, :]).reshape(l, w)
        cand.append(head_sums(prod))
    kc2 = kc_ref[0].reshape(l, 2, w)
    for c in (0, 1):
        cand.append(head_sums(q * kc2[:, c]))

    inv = 1.0 / math.sqrt(D)
    for h in range(NH):
        scores = jnp.concatenate(
            [sc[:, h:h + 1] for sc in cand], axis=-1) * inv

        iota = jax.lax.broadcasted_iota(jnp.int32, (l, KCAND), 1)
        m1 = jnp.min(scores, axis=-1, keepdims=True)
        i1 = jnp.max(jnp.where(scores == m1, iota, -1), axis=-1,
                     keepdims=True)
        drop1 = iota == i1
        s2 = jnp.where(drop1, jnp.inf, scores)
        m2 = jnp.min(s2, axis=-1, keepdims=True)
        i2 = jnp.max(jnp.where(s2 == m2, iota, -1), axis=-1, keepdims=True)
        keep = jnp.logical_not(drop1 | (iota == i2))

        mx = jnp.max(scores, axis=-1, keepdims=True)
        wgt = jnp.where(keep, jnp.exp(scores - mx), 0.0)
        wgt = wgt / jnp.sum(wgt, axis=-1, keepdims=True)

        c0, c1 = h * D, (h + 1) * D
        out = wgt[:, 0:1] * v0_ref[0, 0:l, c0:c1]
        for o in range(1, KBAND):
            out = out + wgt[:, o:o + 1] * v0_ref[0, o:o + l, c0:c1]
        for j, s in enumerate((1, 0, 2)):
            vps = vp_ref[0, s:s + lh, c0:c1]
            vexp = jnp.broadcast_to(vps[:, None, :], (lh, 2, D)).reshape(l, D)
            out = out + wgt[:, KBAND + j:KBAND + j + 1] * vexp
        vc2 = vc_ref[0].reshape(l, 2, w)
        for c in (0, 1):
            out = out + (wgt[:, KBAND + KPAR + c:KBAND + KPAR + c + 1]
                         * vc2[:, c, c0:c1])
        o_ref[0, :, c0:c1] = out


def kernel(x, Wq, Wk, Wv, Wproj, gamma, beta):
    b, l0, c = x.shape
    l = l0 // 2

    rp = 128
    p1, p2, x0 = pl.pallas_call(
        _prep_body,
        grid=(b, (l0 // 4) // rp),
        in_specs=[
            pl.BlockSpec((1, rp, 2, 2, c), lambda bi, i: (bi, i, 0, 0, 0)),
            pl.BlockSpec((1, c), lambda bi, i: (0, 0)),
            pl.BlockSpec((1, c), lambda bi, i: (0, 0)),
        ],
        out_specs=[
            pl.BlockSpec((1, rp, 2, c), lambda bi, i: (bi, i, 0, 0)),
            pl.BlockSpec((1, rp, c), lambda bi, i: (bi, i, 0)),
            pl.BlockSpec((1, rp, 2, c), lambda bi, i: (bi, i, 0, 0)),
        ],
        out_shape=[
            jax.ShapeDtypeStruct((b, l // 2, 2, c), jnp.float32),
            jax.ShapeDtypeStruct((b, l // 2, c), jnp.float32),
            jax.ShapeDtypeStruct((b, l // 2, 2, c), jnp.float32),
        ],
    )(x.reshape(b, l0 // 4, 2, 2, c), gamma.reshape(1, c),
      beta.reshape(1, c))

    wkv = jnp.concatenate([Wk, Wv], axis=1)
    q2d = _matmul(x0.reshape(b * l, c), Wq)
    kv0 = _matmul(p1.reshape(b * l, c), wkv).reshape(b, l, 2 * c)
    kvp = _matmul(p2.reshape(b * l // 2, c), wkv).reshape(b, l // 2, 2 * c)
    kvc = _matmul(x.reshape(b * l0, c), wkv).reshape(b, l0, 2 * c)

    kv0p = jnp.pad(kv0, ((0, 0), (RADIUS, RADIUS), (0, 0)), mode="edge")
    kvpp = jnp.pad(kvp, ((0, 0), (1, 1), (0, 0)), mode="edge")

    ng = H // NH
    wb = NH * D
    attn = pl.pallas_call(
        _attn_body,
        grid=(b, ng),
        in_specs=[
            pl.BlockSpec((1, l, wb), lambda bi, g: (bi, 0, g)),
            pl.BlockSpec((1, l + 2 * RADIUS, wb), lambda bi, g: (bi, 0, g)),
            pl.BlockSpec((1, l + 2 * RADIUS, wb),
                         lambda bi, g: (bi, 0, ng + g)),
            pl.BlockSpec((1, l // 2 + 2, wb), lambda bi, g: (bi, 0, g)),
            pl.BlockSpec((1, l // 2 + 2, wb), lambda bi, g: (bi, 0, ng + g)),
            pl.BlockSpec((1, 2 * l, wb), lambda bi, g: (bi, 0, g)),
            pl.BlockSpec((1, 2 * l, wb), lambda bi, g: (bi, 0, ng + g)),
        ],
        out_specs=pl.BlockSpec((1, l, wb), lambda bi, g: (bi, 0, g)),
        out_shape=jax.ShapeDtypeStruct((b, l, c), jnp.float32),
    )(q2d.reshape(b, l, c), kv0p, kv0p, kvpp, kvpp, kvc, kvc)

    out = _matmul(attn.reshape(b * l, c), Wproj)
    return out.reshape(b, l, c)


# speedup vs baseline: 1.2124x; 1.2124x over previous
import math

import jax
import jax.numpy as jnp
from jax.experimental import pallas as pl

H = 16
D = 64
NH = 2
RADIUS = 6
KBAND = 2 * RADIUS + 1
KPAR = 3
KCHILD = 2
KCAND = KBAND + KPAR + KCHILD


def _prep_body(x_ref, g_ref, b_ref, p1_ref, p2_ref, x0_ref):
    xr = x_ref[0]
    p1b = jnp.max(xr, axis=2)
    p2b = jnp.max(p1b, axis=1)
    m = jnp.mean(p1b, axis=-1, keepdims=True)
    v = jnp.mean((p1b - m) ** 2, axis=-1, keepdims=True)
    x0b = (p1b - m) * jax.lax.rsqrt(v + 1e-5) * g_ref[0] + b_ref[0]
    p1_ref[0] = p1b
    p2_ref[0] = p2b
    x0_ref[0] = x0b


def _matmul_body(a_ref, w_ref, o_ref):
    o_ref[...] = jnp.dot(a_ref[...], w_ref[...],
                         preferred_element_type=jnp.float32)


def _matmul(a, w, bm=512):
    m, k = a.shape
    _, n = w.shape
    return pl.pallas_call(
        _matmul_body,
        grid=(m // bm,),
        in_specs=[
            pl.BlockSpec((bm, k), lambda i: (i, 0)),
            pl.BlockSpec((k, n), lambda i: (0, 0)),
        ],
        out_specs=pl.BlockSpec((bm, n), lambda i: (i, 0)),
        out_shape=jax.ShapeDtypeStruct((m, n), jnp.float32),
    )(a, w)


def _attn_body(q_ref, k0_ref, v0_ref, kp_ref, vp_ref, kc_ref, vc_ref, o_ref):
    q = q_ref[0]
    l, w = q.shape
    lh = l // 2

    row = jax.lax.broadcasted_iota(jnp.int32, (w, NH), 0)
    col = jax.lax.broadcasted_iota(jnp.int32, (w, NH), 1)
    sel = (row // D == col).astype(jnp.float32)

    def head_sums(prod):
        return jnp.dot(prod, sel, preferred_element_type=jnp.float32)

    cand = []
    for o in range(KBAND):
        cand.append(head_sums(q * k0_ref[0, o:o + l]))
    qp = q.reshape(lh, 2, w)
    for s in (1, 0, 2):
        kd = kp_ref[0, s:s + lh]
        prod = (qp * kd[:, None, :]).reshape(l, w)
        cand.append(head_sums(prod))
    kc2 = kc_ref[0].reshape(l, 2, w)
    for c in (0, 1):
        cand.append(head_sums(q * kc2[:, c]))

    inv = 1.0 / math.sqrt(D)
    wgts = []
    for h in range(NH):
        scores = jnp.concatenate(
            [sc[:, h:h + 1] for sc in cand], axis=-1) * inv

        iota = jax.lax.broadcasted_iota(jnp.int32, (l, KCAND), 1)
        m1 = jnp.min(scores, axis=-1, keepdims=True)
        i1 = jnp.max(jnp.where(scores == m1, iota, -1), axis=-1,
                     keepdims=True)
        drop1 = iota == i1
        s2 = jnp.where(drop1, jnp.inf, scores)
        m2 = jnp.min(s2, axis=-1, keepdims=True)
        i2 = jnp.max(jnp.where(s2 == m2, iota, -1), axis=-1, keepdims=True)
        keep = jnp.logical_not(drop1 | (iota == i2))

        mx = jnp.max(scores, axis=-1, keepdims=True)
        wgt = jnp.where(keep, jnp.exp(scores - mx), 0.0)
        wgt = wgt / jnp.sum(wgt, axis=-1, keepdims=True)
        wgts.append(wgt)

    selt = (row // D == col).astype(jnp.float32).T

    def wfull(o):
        pair = jnp.concatenate([wg[:, o:o + 1] for wg in wgts], axis=-1)
        return jnp.dot(pair, selt, preferred_element_type=jnp.float32)

    out = wfull(0) * v0_ref[0, 0:l]
    for o in range(1, KBAND):
        out = out + wfull(o) * v0_ref[0, o:o + l]
    for j, s in enumerate((1, 0, 2)):
        vps = vp_ref[0, s:s + lh]
        vexp = jnp.broadcast_to(vps[:, None, :], (lh, 2, w)).reshape(l, w)
        out = out + wfull(KBAND + j) * vexp
    vc2v = vc_ref[0].reshape(l, 2, w)
    for c in (0, 1):
        out = out + wfull(KBAND + KPAR + c) * vc2v[:, c]
    o_ref[0] = out


def kernel(x, Wq, Wk, Wv, Wproj, gamma, beta):
    b, l0, c = x.shape
    l = l0 // 2

    rp = 128
    p1, p2, x0 = pl.pallas_call(
        _prep_body,
        grid=(b, (l0 // 4) // rp),
        in_specs=[
            pl.BlockSpec((1, rp, 2, 2, c), lambda bi, i: (bi, i, 0, 0, 0)),
            pl.BlockSpec((1, c), lambda bi, i: (0, 0)),
            pl.BlockSpec((1, c), lambda bi, i: (0, 0)),
        ],
        out_specs=[
            pl.BlockSpec((1, rp, 2, c), lambda bi, i: (bi, i, 0, 0)),
            pl.BlockSpec((1, rp, c), lambda bi, i: (bi, i, 0)),
            pl.BlockSpec((1, rp, 2, c), lambda bi, i: (bi, i, 0, 0)),
        ],
        out_shape=[
            jax.ShapeDtypeStruct((b, l // 2, 2, c), jnp.float32),
            jax.ShapeDtypeStruct((b, l // 2, c), jnp.float32),
            jax.ShapeDtypeStruct((b, l // 2, 2, c), jnp.float32),
        ],
    )(x.reshape(b, l0 // 4, 2, 2, c), gamma.reshape(1, c),
      beta.reshape(1, c))

    wkv = jnp.concatenate([Wk, Wv], axis=1)
    q2d = _matmul(x0.reshape(b * l, c), Wq)
    kv0 = _matmul(p1.reshape(b * l, c), wkv).reshape(b, l, 2 * c)
    kvp = _matmul(p2.reshape(b * l // 2, c), wkv).reshape(b, l // 2, 2 * c)
    kvc = _matmul(x.reshape(b * l0, c), wkv).reshape(b, l0, 2 * c)

    kv0p = jnp.pad(kv0, ((0, 0), (RADIUS, RADIUS), (0, 0)), mode="edge")
    kvpp = jnp.pad(kvp, ((0, 0), (1, 1), (0, 0)), mode="edge")

    ng = H // NH
    wb = NH * D
    attn = pl.pallas_call(
        _attn_body,
        grid=(b, ng),
        in_specs=[
            pl.BlockSpec((1, l, wb), lambda bi, g: (bi, 0, g)),
            pl.BlockSpec((1, l + 2 * RADIUS, wb), lambda bi, g: (bi, 0, g)),
            pl.BlockSpec((1, l + 2 * RADIUS, wb),
                         lambda bi, g: (bi, 0, ng + g)),
            pl.BlockSpec((1, l // 2 + 2, wb), lambda bi, g: (bi, 0, g)),
            pl.BlockSpec((1, l // 2 + 2, wb), lambda bi, g: (bi, 0, ng + g)),
            pl.BlockSpec((1, 2 * l, wb), lambda bi, g: (bi, 0, g)),
            pl.BlockSpec((1, 2 * l, wb), lambda bi, g: (bi, 0, ng + g)),
        ],
        out_specs=pl.BlockSpec((1, l, wb), lambda bi, g: (bi, 0, g)),
        out_shape=jax.ShapeDtypeStruct((b, l, c), jnp.float32),
    )(q2d.reshape(b, l, c), kv0p, kv0p, kvpp, kvpp, kvc, kvc)

    out = _matmul(attn.reshape(b * l, c), Wproj)
    return out.reshape(b, l, c)


# speedup vs baseline: 1.7807x; 1.4687x over previous
import math

import jax
import jax.numpy as jnp
from jax.experimental import pallas as pl

H = 16
D = 64
NH = 2
RADIUS = 6
KBAND = 2 * RADIUS + 1
KPAR = 3
KCHILD = 2
PAD = 8
T = 128
WB = T + 2 * PAD
WP = T // 2 + 2 * PAD
WC = 2 * T
WK = WB + WP + WC
EPS = 1e-5


def _prep_body(x_ref, g_ref, b_ref, p1_ref, p2_ref, x0_ref):
    xr = x_ref[0]
    p1b = jnp.max(xr, axis=2)
    p2b = jnp.max(p1b, axis=1)
    m = jnp.mean(p1b, axis=-1, keepdims=True)
    v = jnp.mean((p1b - m) ** 2, axis=-1, keepdims=True)
    x0b = (p1b - m) * jax.lax.rsqrt(v + 1e-5) * g_ref[0] + b_ref[0]
    p1_ref[0] = p1b
    p2_ref[0] = p2b
    x0_ref[0] = x0b


def _matmul_body(a_ref, w_ref, o_ref):
    o_ref[...] = jnp.dot(a_ref[...], w_ref[...],
                         preferred_element_type=jnp.float32)


def _matmul(a, w, bm=512):
    m, k = a.shape
    _, n = w.shape
    return pl.pallas_call(
        _matmul_body,
        grid=(m // bm,),
        in_specs=[
            pl.BlockSpec((bm, k), lambda i: (i, 0)),
            pl.BlockSpec((k, n), lambda i: (0, 0)),
        ],
        out_specs=pl.BlockSpec((bm, n), lambda i: (i, 0)),
        out_shape=jax.ShapeDtypeStruct((m, n), jnp.float32),
    )(a, w)


def _candidate_mask(fill):
    i = jax.lax.broadcasted_iota(jnp.int32, (T, WB), 0)
    j = jax.lax.broadcasted_iota(jnp.int32, (T, WB), 1)
    o = j - i - (PAD - RADIUS)
    mb = jnp.where((o >= 0) & (o <= 2 * RADIUS),
                   -EPS * o.astype(jnp.float32), fill)
    i = jax.lax.broadcasted_iota(jnp.int32, (T, WP), 0)
    j = jax.lax.broadcasted_iota(jnp.int32, (T, WP), 1)
    d = j - PAD - i // 2
    cid = jnp.where(d == 0, KBAND, jnp.where(d == -1, KBAND + 1, KBAND + 2))
    mp = jnp.where((d >= -1) & (d <= 1), -EPS * cid.astype(jnp.float32),
                   fill)
    i = jax.lax.broadcasted_iota(jnp.int32, (T, WC), 0)
    j = jax.lax.broadcasted_iota(jnp.int32, (T, WC), 1)
    c = j - 2 * i
    mc = jnp.where((c >= 0) & (c <= 1),
                   -EPS * (KBAND + KPAR + c).astype(jnp.float32), fill)
    return jnp.concatenate([mb, mp, mc], axis=1)


def _attn_body(q_ref, k0_ref, v0_ref, kp_ref, vp_ref, kc_ref, vc_ref, o_ref):
    l = q_ref.shape[1]
    mask = _candidate_mask(jnp.float32(-jnp.inf))
    big = _candidate_mask(jnp.float32(jnp.inf))
    ones = jnp.ones((WK, D), jnp.float32)
    qs = q_ref[0] * (1.0 / math.sqrt(D))
    nt = (((1,), (1,)), ((), ()))
    nn = (((1,), (0,)), ((), ()))

    for tile in range(l // T):
        i0 = tile * T
        outs = []
        for h in range(NH):
            c0, c1 = h * D, (h + 1) * D
            kcat = jnp.concatenate([
                k0_ref[0, i0:i0 + WB, c0:c1],
                kp_ref[0, i0 // 2:i0 // 2 + WP, c0:c1],
                kc_ref[0, 2 * i0:2 * i0 + WC, c0:c1]], axis=0)
            vcat = jnp.concatenate([
                v0_ref[0, i0:i0 + WB, c0:c1],
                vp_ref[0, i0 // 2:i0 // 2 + WP, c0:c1],
                vc_ref[0, 2 * i0:2 * i0 + WC, c0:c1]], axis=0)
            qk = jax.lax.dot_general(
                qs[i0:i0 + T, c0:c1], kcat, nt,
                preferred_element_type=jnp.float32)
            sm = qk + big
            m1 = jnp.min(sm, axis=-1, keepdims=True)
            m2 = jnp.min(jnp.where(sm == m1, jnp.inf, sm), axis=-1,
                         keepdims=True)
            wp = jnp.where(sm > m2, jnp.exp(qk + mask), 0.0)
            num = jax.lax.dot_general(wp, vcat, nn,
                                      preferred_element_type=jnp.float32)
            den = jax.lax.dot_general(wp, ones, nn,
                                      preferred_element_type=jnp.float32)
            outs.append(num / den)
        o_ref[0, i0:i0 + T] = jnp.concatenate(outs, axis=1)


def kernel(x, Wq, Wk, Wv, Wproj, gamma, beta):
    b, l0, c = x.shape
    l = l0 // 2

    rp = 128
    p1, p2, x0 = pl.pallas_call(
        _prep_body,
        grid=(b, (l0 // 4) // rp),
        in_specs=[
            pl.BlockSpec((1, rp, 2, 2, c), lambda bi, i: (bi, i, 0, 0, 0)),
            pl.BlockSpec((1, c), lambda bi, i: (0, 0)),
            pl.BlockSpec((1, c), lambda bi, i: (0, 0)),
        ],
        out_specs=[
            pl.BlockSpec((1, rp, 2, c), lambda bi, i: (bi, i, 0, 0)),
            pl.BlockSpec((1, rp, c), lambda bi, i: (bi, i, 0)),
            pl.BlockSpec((1, rp, 2, c), lambda bi, i: (bi, i, 0, 0)),
        ],
        out_shape=[
            jax.ShapeDtypeStruct((b, l // 2, 2, c), jnp.float32),
            jax.ShapeDtypeStruct((b, l // 2, c), jnp.float32),
            jax.ShapeDtypeStruct((b, l // 2, 2, c), jnp.float32),
        ],
    )(x.reshape(b, l0 // 4, 2, 2, c), gamma.reshape(1, c),
      beta.reshape(1, c))

    wkv = jnp.concatenate([Wk, Wv], axis=1)
    q2d = _matmul(x0.reshape(b * l, c), Wq)
    kv0 = _matmul(p1.reshape(b * l, c), wkv).reshape(b, l, 2 * c)
    kvp = _matmul(p2.reshape(b * l // 2, c), wkv).reshape(b, l // 2, 2 * c)
    kvc = _matmul(x.reshape(b * l0, c), wkv).reshape(b, l0, 2 * c)

    kv0p = jnp.pad(kv0, ((0, 0), (PAD, PAD), (0, 0)), mode="edge")
    kvpp = jnp.pad(kvp, ((0, 0), (PAD, PAD), (0, 0)), mode="edge")

    ng = H // NH
    wb = NH * D
    attn = pl.pallas_call(
        _attn_body,
        grid=(b, ng),
        in_specs=[
            pl.BlockSpec((1, l, wb), lambda bi, g: (bi, 0, g)),
            pl.BlockSpec((1, l + 2 * PAD, wb), lambda bi, g: (bi, 0, g)),
            pl.BlockSpec((1, l + 2 * PAD, wb), lambda bi, g: (bi, 0, ng + g)),
            pl.BlockSpec((1, l // 2 + 2 * PAD, wb),
                         lambda bi, g: (bi, 0, g)),
            pl.BlockSpec((1, l // 2 + 2 * PAD, wb),
                         lambda bi, g: (bi, 0, ng + g)),
            pl.BlockSpec((1, 2 * l, wb), lambda bi, g: (bi, 0, g)),
            pl.BlockSpec((1, 2 * l, wb), lambda bi, g: (bi, 0, ng + g)),
        ],
        out_specs=pl.BlockSpec((1, l, wb), lambda bi, g: (bi, 0, g)),
        out_shape=jax.ShapeDtypeStruct((b, l, c), jnp.float32),
    )(q2d.reshape(b, l, c), kv0p, kv0p, kvpp, kvpp, kvc, kvc)

    out = _matmul(attn.reshape(b * l, c), Wproj)
    return out.reshape(b, l, c)


# speedup vs baseline: 1.7812x; 1.0003x over previous
import math

import jax
import jax.numpy as jnp
from jax.experimental import pallas as pl
from jax.experimental.pallas import tpu as pltpu

H = 16
D = 64
NH = 2
RADIUS = 6
KBAND = 2 * RADIUS + 1
KPAR = 3
KCHILD = 2
PAD = 8
T = 128
WB = T + 2 * PAD
WP = T // 2 + 2 * PAD
WC = 2 * T
WK = WB + WP + WC
EPS = 1e-5


def _prep_body(x_ref, g_ref, b_ref, p1_ref, p2_ref, x0_ref):
    xr = x_ref[0]
    p1b = jnp.max(xr, axis=2)
    p2b = jnp.max(p1b, axis=1)
    m = jnp.mean(p1b, axis=-1, keepdims=True)
    v = jnp.mean((p1b - m) ** 2, axis=-1, keepdims=True)
    x0b = (p1b - m) * jax.lax.rsqrt(v + 1e-5) * g_ref[0] + b_ref[0]
    p1_ref[0] = p1b
    p2_ref[0] = p2b
    x0_ref[0] = x0b


def _matmul_body(a_ref, w_ref, o_ref):
    o_ref[...] = jnp.dot(a_ref[...], w_ref[...],
                         preferred_element_type=jnp.float32)


def _matmul(a, w, bm=512):
    m, k = a.shape
    _, n = w.shape
    return pl.pallas_call(
        _matmul_body,
        grid=(m // bm,),
        in_specs=[
            pl.BlockSpec((bm, k), lambda i: (i, 0)),
            pl.BlockSpec((k, n), lambda i: (0, 0)),
        ],
        out_specs=pl.BlockSpec((bm, n), lambda i: (i, 0)),
        out_shape=jax.ShapeDtypeStruct((m, n), jnp.float32),
        compiler_params=pltpu.CompilerParams(
            dimension_semantics=("parallel",)),
    )(a, w)


def _candidate_mask(fill):
    i = jax.lax.broadcasted_iota(jnp.int32, (T, WB), 0)
    j = jax.lax.broadcasted_iota(jnp.int32, (T, WB), 1)
    o = j - i - (PAD - RADIUS)
    mb = jnp.where((o >= 0) & (o <= 2 * RADIUS),
                   -EPS * o.astype(jnp.float32), fill)
    i = jax.lax.broadcasted_iota(jnp.int32, (T, WP), 0)
    j = jax.lax.broadcasted_iota(jnp.int32, (T, WP), 1)
    d = j - PAD - i // 2
    cid = jnp.where(d == 0, KBAND, jnp.where(d == -1, KBAND + 1, KBAND + 2))
    mp = jnp.where((d >= -1) & (d <= 1), -EPS * cid.astype(jnp.float32),
                   fill)
    i = jax.lax.broadcasted_iota(jnp.int32, (T, WC), 0)
    j = jax.lax.broadcasted_iota(jnp.int32, (T, WC), 1)
    c = j - 2 * i
    mc = jnp.where((c >= 0) & (c <= 1),
                   -EPS * (KBAND + KPAR + c).astype(jnp.float32), fill)
    return jnp.concatenate([mb, mp, mc], axis=1)


def _attn_body(q_ref, k0_ref, v0_ref, kp_ref, vp_ref, kc_ref, vc_ref, o_ref):
    l = q_ref.shape[1]
    mask = _candidate_mask(jnp.float32(-jnp.inf))
    big = _candidate_mask(jnp.float32(jnp.inf))
    ones = jnp.ones((WK, D), jnp.float32)
    qs = q_ref[0] * (1.0 / math.sqrt(D))
    nt = (((1,), (1,)), ((), ()))
    nn = (((1,), (0,)), ((), ()))

    for tile in range(l // T):
        i0 = tile * T
        outs = []
        for h in range(NH):
            c0, c1 = h * D, (h + 1) * D
            kcat = jnp.concatenate([
                k0_ref[0, i0:i0 + WB, c0:c1],
                kp_ref[0, i0 // 2:i0 // 2 + WP, c0:c1],
                kc_ref[0, 2 * i0:2 * i0 + WC, c0:c1]], axis=0)
            vcat = jnp.concatenate([
                v0_ref[0, i0:i0 + WB, c0:c1],
                vp_ref[0, i0 // 2:i0 // 2 + WP, c0:c1],
                vc_ref[0, 2 * i0:2 * i0 + WC, c0:c1]], axis=0)
            qk = jax.lax.dot_general(
                qs[i0:i0 + T, c0:c1], kcat, nt,
                preferred_element_type=jnp.float32)
            sm = qk + big
            m1 = jnp.min(sm, axis=-1, keepdims=True)
            m2 = jnp.min(jnp.where(sm == m1, jnp.inf, sm), axis=-1,
                         keepdims=True)
            wp = jnp.where(sm > m2, jnp.exp(qk + mask), 0.0)
            num = jax.lax.dot_general(wp, vcat, nn,
                                      preferred_element_type=jnp.float32)
            den = jax.lax.dot_general(wp, ones, nn,
                                      preferred_element_type=jnp.float32)
            outs.append(num / den)
        o_ref[0, i0:i0 + T] = jnp.concatenate(outs, axis=1)


def kernel(x, Wq, Wk, Wv, Wproj, gamma, beta):
    b, l0, c = x.shape
    l = l0 // 2

    rp = 128
    p1, p2, x0 = pl.pallas_call(
        _prep_body,
        grid=(b, (l0 // 4) // rp),
        in_specs=[
            pl.BlockSpec((1, rp, 2, 2, c), lambda bi, i: (bi, i, 0, 0, 0)),
            pl.BlockSpec((1, c), lambda bi, i: (0, 0)),
            pl.BlockSpec((1, c), lambda bi, i: (0, 0)),
        ],
        out_specs=[
            pl.BlockSpec((1, rp, 2, c), lambda bi, i: (bi, i, 0, 0)),
            pl.BlockSpec((1, rp, c), lambda bi, i: (bi, i, 0)),
            pl.BlockSpec((1, rp, 2, c), lambda bi, i: (bi, i, 0, 0)),
        ],
        out_shape=[
            jax.ShapeDtypeStruct((b, l // 2, 2, c), jnp.float32),
            jax.ShapeDtypeStruct((b, l // 2, c), jnp.float32),
            jax.ShapeDtypeStruct((b, l // 2, 2, c), jnp.float32),
        ],
        compiler_params=pltpu.CompilerParams(
            dimension_semantics=("parallel", "parallel")),
    )(x.reshape(b, l0 // 4, 2, 2, c), gamma.reshape(1, c),
      beta.reshape(1, c))

    wkv = jnp.concatenate([Wk, Wv], axis=1)
    q2d = _matmul(x0.reshape(b * l, c), Wq)
    kv0 = _matmul(p1.reshape(b * l, c), wkv).reshape(b, l, 2 * c)
    kvp = _matmul(p2.reshape(b * l // 2, c), wkv).reshape(b, l // 2, 2 * c)
    kvc = _matmul(x.reshape(b * l0, c), wkv).reshape(b, l0, 2 * c)

    kv0p = jnp.pad(kv0, ((0, 0), (PAD, PAD), (0, 0)), mode="edge")
    kvpp = jnp.pad(kvp, ((0, 0), (PAD, PAD), (0, 0)), mode="edge")

    ng = H // NH
    wb = NH * D
    attn = pl.pallas_call(
        _attn_body,
        grid=(b, ng),
        in_specs=[
            pl.BlockSpec((1, l, wb), lambda bi, g: (bi, 0, g)),
            pl.BlockSpec((1, l + 2 * PAD, wb), lambda bi, g: (bi, 0, g)),
            pl.BlockSpec((1, l + 2 * PAD, wb), lambda bi, g: (bi, 0, ng + g)),
            pl.BlockSpec((1, l // 2 + 2 * PAD, wb),
                         lambda bi, g: (bi, 0, g)),
            pl.BlockSpec((1, l // 2 + 2 * PAD, wb),
                         lambda bi, g: (bi, 0, ng + g)),
            pl.BlockSpec((1, 2 * l, wb), lambda bi, g: (bi, 0, g)),
            pl.BlockSpec((1, 2 * l, wb), lambda bi, g: (bi, 0, ng + g)),
        ],
        out_specs=pl.BlockSpec((1, l, wb), lambda bi, g: (bi, 0, g)),
        out_shape=jax.ShapeDtypeStruct((b, l, c), jnp.float32),
        compiler_params=pltpu.CompilerParams(
            dimension_semantics=("parallel", "parallel")),
    )(q2d.reshape(b, l, c), kv0p, kv0p, kvpp, kvpp, kvc, kvc)

    out = _matmul(attn.reshape(b * l, c), Wproj)
    return out.reshape(b, l, c)


# speedup vs baseline: 2.0786x; 1.1670x over previous
import math

import jax
import jax.numpy as jnp
from jax.experimental import pallas as pl
from jax.experimental.pallas import tpu as pltpu

H = 16
D = 64
NH = 2
RADIUS = 6
KBAND = 2 * RADIUS + 1
KPAR = 3
KCHILD = 2
PAD = 8
T = 128
WB = T + 2 * PAD
WP = T // 2 + 2 * PAD
WC = 2 * T
WK = WB + WP + WC
EPS = 1e-5


def _prep_body(x2_ref, x4_ref, g_ref, b_ref, p1_ref, p2_ref, x0_ref):
    c = p1_ref.shape[-1]
    x2 = x2_ref[0]
    x4 = x4_ref[0]
    p1b = jnp.maximum(x2[:, :c], x2[:, c:])
    p2b = jnp.maximum(jnp.maximum(x4[:, :c], x4[:, c:2 * c]),
                      jnp.maximum(x4[:, 2 * c:3 * c], x4[:, 3 * c:]))
    m = jnp.mean(p1b, axis=-1, keepdims=True)
    v = jnp.mean((p1b - m) ** 2, axis=-1, keepdims=True)
    x0b = (p1b - m) * jax.lax.rsqrt(v + 1e-5) * g_ref[0] + b_ref[0]
    p1_ref[0] = p1b
    p2_ref[0] = p2b
    x0_ref[0] = x0b


def _matmul_body(a_ref, w_ref, o_ref):
    o_ref[...] = jnp.dot(a_ref[...], w_ref[...],
                         preferred_element_type=jnp.float32)


def _matmul(a, w, bm=512):
    m, k = a.shape
    _, n = w.shape
    return pl.pallas_call(
        _matmul_body,
        grid=(m // bm,),
        in_specs=[
            pl.BlockSpec((bm, k), lambda i: (i, 0)),
            pl.BlockSpec((k, n), lambda i: (0, 0)),
        ],
        out_specs=pl.BlockSpec((bm, n), lambda i: (i, 0)),
        out_shape=jax.ShapeDtypeStruct((m, n), jnp.float32),
        compiler_params=pltpu.CompilerParams(
            dimension_semantics=("parallel",)),
    )(a, w)


def _candidate_mask(fill):
    i = jax.lax.broadcasted_iota(jnp.int32, (T, WB), 0)
    j = jax.lax.broadcasted_iota(jnp.int32, (T, WB), 1)
    o = j - i - (PAD - RADIUS)
    mb = jnp.where((o >= 0) & (o <= 2 * RADIUS),
                   -EPS * o.astype(jnp.float32), fill)
    i = jax.lax.broadcasted_iota(jnp.int32, (T, WP), 0)
    j = jax.lax.broadcasted_iota(jnp.int32, (T, WP), 1)
    d = j - PAD - i // 2
    cid = jnp.where(d == 0, KBAND, jnp.where(d == -1, KBAND + 1, KBAND + 2))
    mp = jnp.where((d >= -1) & (d <= 1), -EPS * cid.astype(jnp.float32),
                   fill)
    i = jax.lax.broadcasted_iota(jnp.int32, (T, WC), 0)
    j = jax.lax.broadcasted_iota(jnp.int32, (T, WC), 1)
    c = j - 2 * i
    mc = jnp.where((c >= 0) & (c <= 1),
                   -EPS * (KBAND + KPAR + c).astype(jnp.float32), fill)
    return jnp.concatenate([mb, mp, mc], axis=1)


def _attn_body(q_ref, k0_ref, v0_ref, kp_ref, vp_ref, kc_ref, vc_ref, o_ref):
    l = q_ref.shape[1]
    mask = _candidate_mask(jnp.float32(-jnp.inf))
    big = _candidate_mask(jnp.float32(jnp.inf))
    ones = jnp.ones((WK, D), jnp.float32)
    qs = q_ref[0] * (1.0 / math.sqrt(D))
    nt = (((1,), (1,)), ((), ()))
    nn = (((1,), (0,)), ((), ()))

    for tile in range(l // T):
        i0 = tile * T
        outs = []
        for h in range(NH):
            c0, c1 = h * D, (h + 1) * D
            kcat = jnp.concatenate([
                k0_ref[0, i0:i0 + WB, c0:c1],
                kp_ref[0, i0 // 2:i0 // 2 + WP, c0:c1],
                kc_ref[0, 2 * i0:2 * i0 + WC, c0:c1]], axis=0)
            vcat = jnp.concatenate([
                v0_ref[0, i0:i0 + WB, c0:c1],
                vp_ref[0, i0 // 2:i0 // 2 + WP, c0:c1],
                vc_ref[0, 2 * i0:2 * i0 + WC, c0:c1]], axis=0)
            qk = jax.lax.dot_general(
                qs[i0:i0 + T, c0:c1], kcat, nt,
                preferred_element_type=jnp.float32)
            sm = qk + big
            m1 = jnp.min(sm, axis=-1, keepdims=True)
            m2 = jnp.min(jnp.where(sm == m1, jnp.inf, sm), axis=-1,
                         keepdims=True)
            wp = jnp.where(sm > m2, jnp.exp(qk + mask), 0.0)
            num = jax.lax.dot_general(wp, vcat, nn,
                                      preferred_element_type=jnp.float32)
            den = jax.lax.dot_general(wp, ones, nn,
                                      preferred_element_type=jnp.float32)
            outs.append(num / den)
        o_ref[0, i0:i0 + T] = jnp.concatenate(outs, axis=1)


def kernel(x, Wq, Wk, Wv, Wproj, gamma, beta):
    b, l0, c = x.shape
    l = l0 // 2

    rp = 256
    p1, p2, x0 = pl.pallas_call(
        _prep_body,
        grid=(b, l // rp),
        in_specs=[
            pl.BlockSpec((1, rp, 2 * c), lambda bi, i: (bi, i, 0)),
            pl.BlockSpec((1, rp // 2, 4 * c), lambda bi, i: (bi, i, 0)),
            pl.BlockSpec((1, c), lambda bi, i: (0, 0)),
            pl.BlockSpec((1, c), lambda bi, i: (0, 0)),
        ],
        out_specs=[
            pl.BlockSpec((1, rp, c), lambda bi, i: (bi, i, 0)),
            pl.BlockSpec((1, rp // 2, c), lambda bi, i: (bi, i, 0)),
            pl.BlockSpec((1, rp, c), lambda bi, i: (bi, i, 0)),
        ],
        out_shape=[
            jax.ShapeDtypeStruct((b, l, c), jnp.float32),
            jax.ShapeDtypeStruct((b, l // 2, c), jnp.float32),
            jax.ShapeDtypeStruct((b, l, c), jnp.float32),
        ],
        compiler_params=pltpu.CompilerParams(
            dimension_semantics=("parallel", "parallel")),
    )(x.reshape(b, l, 2 * c), x.reshape(b, l // 2, 4 * c),
      gamma.reshape(1, c), beta.reshape(1, c))

    wkv = jnp.concatenate([Wk, Wv], axis=1)
    q2d = _matmul(x0.reshape(b * l, c), Wq)
    kv0 = _matmul(p1.reshape(b * l, c), wkv).reshape(b, l, 2 * c)
    kvp = _matmul(p2.reshape(b * l // 2, c), wkv).reshape(b, l // 2, 2 * c)
    kvc = _matmul(x.reshape(b * l0, c), wkv).reshape(b, l0, 2 * c)

    kv0p = jnp.pad(kv0, ((0, 0), (PAD, PAD), (0, 0)), mode="edge")
    kvpp = jnp.pad(kvp, ((0, 0), (PAD, PAD), (0, 0)), mode="edge")

    ng = H // NH
    wb = NH * D
    attn = pl.pallas_call(
        _attn_body,
        grid=(b, ng),
        in_specs=[
            pl.BlockSpec((1, l, wb), lambda bi, g: (bi, 0, g)),
            pl.BlockSpec((1, l + 2 * PAD, wb), lambda bi, g: (bi, 0, g)),
            pl.BlockSpec((1, l + 2 * PAD, wb), lambda bi, g: (bi, 0, ng + g)),
            pl.BlockSpec((1, l // 2 + 2 * PAD, wb),
                         lambda bi, g: (bi, 0, g)),
            pl.BlockSpec((1, l // 2 + 2 * PAD, wb),
                         lambda bi, g: (bi, 0, ng + g)),
            pl.BlockSpec((1, 2 * l, wb), lambda bi, g: (bi, 0, g)),
            pl.BlockSpec((1, 2 * l, wb), lambda bi, g: (bi, 0, ng + g)),
        ],
        out_specs=pl.BlockSpec((1, l, wb), lambda bi, g: (bi, 0, g)),
        out_shape=jax.ShapeDtypeStruct((b, l, c), jnp.float32),
        compiler_params=pltpu.CompilerParams(
            dimension_semantics=("parallel", "parallel")),
    )(q2d.reshape(b, l, c), kv0p, kv0p, kvpp, kvpp, kvc, kvc)

    out = _matmul(attn.reshape(b * l, c), Wproj)
    return out.reshape(b, l, c)


# speedup vs baseline: 2.2322x; 1.0739x over previous
import math

import jax
import jax.numpy as jnp
from jax.experimental import pallas as pl
from jax.experimental.pallas import tpu as pltpu

H = 16
D = 64
NH = 2
RADIUS = 6
KBAND = 2 * RADIUS + 1
KPAR = 3
KCHILD = 2
PAD = 8
T = 128
WB = T + 2 * PAD
WP = T // 2 + 2 * PAD
WC = 2 * T
WK = WB + WP + WC
EPS = 1e-5


def _prep_body(x2_ref, x4_ref, g_ref, b_ref, p1_ref, p2_ref, x0_ref):
    c = p1_ref.shape[-1]
    x2 = x2_ref[0]
    x4 = x4_ref[0]
    p1b = jnp.maximum(x2[:, :c], x2[:, c:])
    p2b = jnp.maximum(jnp.maximum(x4[:, :c], x4[:, c:2 * c]),
                      jnp.maximum(x4[:, 2 * c:3 * c], x4[:, 3 * c:]))
    m = jnp.mean(p1b, axis=-1, keepdims=True)
    v = jnp.mean((p1b - m) ** 2, axis=-1, keepdims=True)
    x0b = (p1b - m) * jax.lax.rsqrt(v + 1e-5) * g_ref[0] + b_ref[0]
    p1_ref[0] = p1b
    p2_ref[0] = p2b
    x0_ref[0] = x0b


def _matmul_body(a_ref, w_ref, o_ref):
    o_ref[...] = jnp.dot(a_ref[...], w_ref[...],
                         preferred_element_type=jnp.float32)


def _matmul(a, w, bm=512):
    m, k = a.shape
    _, n = w.shape
    return pl.pallas_call(
        _matmul_body,
        grid=(m // bm,),
        in_specs=[
            pl.BlockSpec((bm, k), lambda i: (i, 0)),
            pl.BlockSpec((k, n), lambda i: (0, 0)),
        ],
        out_specs=pl.BlockSpec((bm, n), lambda i: (i, 0)),
        out_shape=jax.ShapeDtypeStruct((m, n), jnp.float32),
        compiler_params=pltpu.CompilerParams(
            dimension_semantics=("parallel",)),
    )(a, w)


def _candidate_mask(fill):
    i = jax.lax.broadcasted_iota(jnp.int32, (T, WB), 0)
    j = jax.lax.broadcasted_iota(jnp.int32, (T, WB), 1)
    o = j - i - (PAD - RADIUS)
    mb = jnp.where((o >= 0) & (o <= 2 * RADIUS),
                   -EPS * o.astype(jnp.float32), fill)
    i = jax.lax.broadcasted_iota(jnp.int32, (T, WP), 0)
    j = jax.lax.broadcasted_iota(jnp.int32, (T, WP), 1)
    d = j - PAD - i // 2
    cid = jnp.where(d == 0, KBAND, jnp.where(d == -1, KBAND + 1, KBAND + 2))
    mp = jnp.where((d >= -1) & (d <= 1), -EPS * cid.astype(jnp.float32),
                   fill)
    i = jax.lax.broadcasted_iota(jnp.int32, (T, WC), 0)
    j = jax.lax.broadcasted_iota(jnp.int32, (T, WC), 1)
    c = j - 2 * i
    mc = jnp.where((c >= 0) & (c <= 1),
                   -EPS * (KBAND + KPAR + c).astype(jnp.float32), fill)
    return jnp.concatenate([mb, mp, mc], axis=1)


def _attn_body(q_ref, k0_ref, v0_ref, kp_ref, vp_ref, kc_ref, vc_ref, o_ref):
    l = q_ref.shape[1]
    mask = _candidate_mask(jnp.float32(-jnp.inf))
    big = _candidate_mask(jnp.float32(jnp.inf))
    ones = jnp.ones((WK, D), jnp.float32)
    qs = q_ref[0] * (1.0 / math.sqrt(D))
    nt = (((1,), (1,)), ((), ()))
    nn = (((1,), (0,)), ((), ()))

    for tile in range(l // T):
        i0 = tile * T
        outs = []
        for h in range(NH):
            c0, c1 = h * D, (h + 1) * D
            kcat = jnp.concatenate([
                k0_ref[0, i0:i0 + WB, c0:c1],
                kp_ref[0, i0 // 2:i0 // 2 + WP, c0:c1],
                kc_ref[0, 2 * i0:2 * i0 + WC, c0:c1]], axis=0)
            vcat = jnp.concatenate([
                v0_ref[0, i0:i0 + WB, c0:c1],
                vp_ref[0, i0 // 2:i0 // 2 + WP, c0:c1],
                vc_ref[0, 2 * i0:2 * i0 + WC, c0:c1]], axis=0)
            qk = jax.lax.dot_general(
                qs[i0:i0 + T, c0:c1], kcat, nt,
                preferred_element_type=jnp.float32)
            sm = qk + big
            m1 = jnp.min(sm, axis=-1, keepdims=True)
            m2 = jnp.min(jnp.where(sm == m1, jnp.inf, sm), axis=-1,
                         keepdims=True)
            wp = jnp.where(sm > m2, jnp.exp(qk + mask), 0.0)
            num = jax.lax.dot_general(wp, vcat, nn,
                                      preferred_element_type=jnp.float32)
            den = jax.lax.dot_general(wp, ones, nn,
                                      preferred_element_type=jnp.float32)
            outs.append(num / den)
        o_ref[0, i0:i0 + T] = jnp.concatenate(outs, axis=1)


def kernel(x, Wq, Wk, Wv, Wproj, gamma, beta):
    b, l0, c = x.shape
    l = l0 // 2

    rp = 256
    p1, p2, x0 = pl.pallas_call(
        _prep_body,
        grid=(b, l // rp),
        in_specs=[
            pl.BlockSpec((1, rp, 2 * c), lambda bi, i: (bi, i, 0)),
            pl.BlockSpec((1, rp // 2, 4 * c), lambda bi, i: (bi, i, 0)),
            pl.BlockSpec((1, c), lambda bi, i: (0, 0)),
            pl.BlockSpec((1, c), lambda bi, i: (0, 0)),
        ],
        out_specs=[
            pl.BlockSpec((1, rp, c), lambda bi, i: (bi, i, 0)),
            pl.BlockSpec((1, rp // 2, c), lambda bi, i: (bi, i, 0)),
            pl.BlockSpec((1, rp, c), lambda bi, i: (bi, i, 0)),
        ],
        out_shape=[
            jax.ShapeDtypeStruct((b, l, c), jnp.float32),
            jax.ShapeDtypeStruct((b, l // 2, c), jnp.float32),
            jax.ShapeDtypeStruct((b, l, c), jnp.float32),
        ],
        compiler_params=pltpu.CompilerParams(
            dimension_semantics=("parallel", "parallel")),
    )(x.reshape(b, l, 2 * c), x.reshape(b, l // 2, 4 * c),
      gamma.reshape(1, c), beta.reshape(1, c))

    wkv = jnp.concatenate([Wk, Wv], axis=1)
    q2d = _matmul(x0.reshape(b * l, c), Wq)
    kv0 = _matmul(p1.reshape(b * l, c), wkv).reshape(b, l, 2 * c)
    kvp = _matmul(p2.reshape(b * l // 2, c), wkv).reshape(b, l // 2, 2 * c)
    kvc = _matmul(x.reshape(b * l0, c), wkv).reshape(b, l0, 2 * c)

    kv0p = jnp.pad(kv0, ((0, 0), (PAD, PAD), (0, 0)), mode="edge")
    kvpp = jnp.pad(kvp, ((0, 0), (PAD, PAD), (0, 0)), mode="edge")

    ng = H // NH
    wb = NH * D
    attn = pl.pallas_call(
        _attn_body,
        grid=(b, ng),
        in_specs=[
            pl.BlockSpec((1, l, wb), lambda bi, g: (bi, 0, g)),
            pl.BlockSpec((1, l + 2 * PAD, wb), lambda bi, g: (bi, 0, g)),
            pl.BlockSpec((1, l + 2 * PAD, wb), lambda bi, g: (bi, 0, ng + g)),
            pl.BlockSpec((1, l // 2 + 2 * PAD, wb),
                         lambda bi, g: (bi, 0, g)),
            pl.BlockSpec((1, l // 2 + 2 * PAD, wb),
                         lambda bi, g: (bi, 0, ng + g)),
            pl.BlockSpec((1, 2 * l, wb), lambda bi, g: (bi, 0, g)),
            pl.BlockSpec((1, 2 * l, wb), lambda bi, g: (bi, 0, ng + g)),
        ],
        out_specs=pl.BlockSpec((1, l, wb), lambda bi, g: (bi, 0, g)),
        out_shape=jax.ShapeDtypeStruct((b, l, c), jnp.float32),
        compiler_params=pltpu.CompilerParams(
            dimension_semantics=("parallel", "parallel"),
            allow_input_fusion=[False, True, True, True, True, False, False]),
    )(q2d.reshape(b, l, c), kv0p, kv0p, kvpp, kvpp, kvc, kvc)

    out = _matmul(attn.reshape(b * l, c), Wproj)
    return out.reshape(b, l, c)
